# baseline (speedup 1.0000x reference)
import jax
import jax.numpy as jnp
from jax.experimental import pallas as pl
from jax.experimental.pallas import tpu as pltpu

N = 4096
D = 512
H = 128
T = 128
NT = N // T
THR = 0.9
_BN_SCALE = 1.0 / (1.0 + 1e-5) ** 0.5


def _stats_kernel(x_ref, xn_ref, dinv_ref, cnt_ref):
    p = pl.program_id(0)
    i = pl.program_id(1)
    rows = pl.ds(i * T, T)

    @pl.when(p == 0)
    def _():
        xi = x_ref[...]
        nrm = jnp.sqrt(jnp.sum(xi * xi, axis=1, keepdims=True))
        xn_ref[rows, :] = xi / jnp.maximum(nrm, 1e-12)

    @pl.when(p == 1)
    def _():
        xi = xn_ref[rows, :]
        s = jax.lax.dot_general(xi, xn_ref[...], (((1,), (1,)), ((), ())),
                                preferred_element_type=jnp.float32)
        col = jax.lax.broadcasted_iota(jnp.int32, (T, N), 1)
        row = jax.lax.broadcasted_iota(jnp.int32, (T, N), 0) + i * T
        keep = (s > THR) & (col != row)
        a = jnp.where(keep, s, 0.0)
        deg = jnp.sum(a, axis=1, keepdims=True)
        dinv_ref[rows, :] = jnp.where(
            deg > 0, jax.lax.rsqrt(jnp.maximum(deg, 1e-12)), 0.0)
        colnz = jnp.sum(keep.astype(jnp.float32), axis=0, keepdims=True)
        bm_r = jax.lax.broadcasted_iota(jnp.int32, (N, NT), 0) // T
        bm_c = jax.lax.broadcasted_iota(jnp.int32, (N, NT), 1)
        bm = (bm_r == bm_c).astype(jnp.float32)
        cnt_ref[pl.ds(i, 1), :] = jax.lax.dot_general(
            colnz, bm, (((1,), (0,)), ((), ())),
            preferred_element_type=jnp.float32)


def _gcn_kernel(x_ref, xn_ref, dinv_ref, cnt_ref,
                w00, w01, w02, w10, w11, w12, w20, w21, w22,
                l1w, l1b, bg, bb, l2w, l2b,
                out_ref,
                v1_s, v2_s, base_s, h1_s, h2_s, h3_s, acc_s):
    p = pl.program_id(0)
    i = pl.program_id(1)
    rows = pl.ds(i * T, T)
    dv = dinv_ref[rows, :]

    def phase_a(h, wa, wb, wc):
        b0 = jnp.dot(h, wa[...], preferred_element_type=jnp.float32)
        v1 = jnp.dot(h, wb[...], preferred_element_type=jnp.float32)
        v2 = jnp.dot(h, wc[...], preferred_element_type=jnp.float32)
        base_s[rows, :] = b0 - v2
        v1_s[rows, :] = v1
        v2_s[rows, :] = dv * v2

    def spmm_rows(src_s):
        acc_s[...] = jnp.zeros((T, H), jnp.float32)
        xi = xn_ref[rows, :]

        def body(j, carry):
            @pl.when(cnt_ref[i, j] > 0)
            def _():
                xj = xn_ref[pl.ds(j * T, T), :]
                s = jax.lax.dot_general(xi, xj, (((1,), (1,)), ((), ())),
                                        preferred_element_type=jnp.float32)
                colg = jax.lax.broadcasted_iota(jnp.int32, (T, T), 1) + j * T
                rowg = jax.lax.broadcasted_iota(jnp.int32, (T, T), 0) + i * T
                a = jnp.where((s > THR) & (colg != rowg), s, 0.0)
                acc_s[...] += jnp.dot(a, src_s[pl.ds(j * T, T), :],
                                      preferred_element_type=jnp.float32)
            return carry

        jax.lax.fori_loop(0, NT, body, 0)
        return -dv * acc_s[...]

    def phase_b():
        m2 = spmm_rows(v2_s)
        v1_s[rows, :] = dv * (v1_s[rows, :] + 2.0 * m2)

    def phase_c(hout_s):
        t = spmm_rows(v1_s)
        hout_s[rows, :] = jnp.maximum(base_s[rows, :] + t, 0.0)

    @pl.when(p == 0)
    def _():
        phase_a(x_ref[...], w00, w01, w02)

    @pl.when(p == 1)
    def _():
        phase_b()

    @pl.when(p == 2)
    def _():
        phase_c(h1_s)

    @pl.when(p == 3)
    def _():
        phase_a(h1_s[rows, :], w10, w11, w12)

    @pl.when(p == 4)
    def _():
        phase_b()

    @pl.when(p == 5)
    def _():
        phase_c(h2_s)

    @pl.when(p == 6)
    def _():
        phase_a(h2_s[rows, :], w20, w21, w22)

    @pl.when(p == 7)
    def _():
        phase_b()

    @pl.when(p == 8)
    def _():
        phase_c(h3_s)

    @pl.when(p == 9)
    def _():
        jk = jnp.concatenate(
            [h1_s[rows, :], h2_s[rows, :], h3_s[rows, :]], axis=1)
        z = jnp.dot(jk, l1w[...], preferred_element_type=jnp.float32) + l1b[...]
        z = jnp.maximum(z, 0.0)
        z = z * (_BN_SCALE * bg[...]) + bb[...]
        logit = jnp.dot(z, l2w[...], preferred_element_type=jnp.float32) + l2b[...]
        m = jnp.max(logit, axis=1, keepdims=True)
        e = jnp.exp(logit - m)
        out_ref[rows, :] = e / jnp.sum(e, axis=1, keepdims=True)


def kernel(x, w0_0, w0_1, w0_2, w1_0, w1_1, w1_2, w2_0, w2_1, w2_2,
           lin1_w, lin1_b, bn_gamma, bn_beta, lin2_w, lin2_b):
    f32 = jnp.float32
    xn, dinv, cntf = pl.pallas_call(
        _stats_kernel,
        grid=(2, NT),
        in_specs=[pl.BlockSpec((T, D), lambda p, i: (i, 0))],
        out_specs=[
            pl.BlockSpec((N, D), lambda p, i: (0, 0)),
            pl.BlockSpec((N, 1), lambda p, i: (0, 0)),
            pl.BlockSpec((NT, NT), lambda p, i: (0, 0)),
        ],
        out_shape=[
            jax.ShapeDtypeStruct((N, D), f32),
            jax.ShapeDtypeStruct((N, 1), f32),
            jax.ShapeDtypeStruct((NT, NT), f32),
        ],
        compiler_params=pltpu.CompilerParams(
            dimension_semantics=("arbitrary", "arbitrary")),
    )(x)
    cnt = cntf.astype(jnp.int32)

    full = pl.BlockSpec(memory_space=pltpu.VMEM)
    out = pl.pallas_call(
        _gcn_kernel,
        grid=(10, NT),
        in_specs=[pl.BlockSpec((T, D), lambda p, i: (i, 0)),
                  full,
                  full,
                  pl.BlockSpec(memory_space=pltpu.SMEM)]
                 + [full] * 15,
        out_specs=pl.BlockSpec((N, 10), lambda p, i: (0, 0)),
        out_shape=jax.ShapeDtypeStruct((N, 10), f32),
        scratch_shapes=[pltpu.VMEM((N, H), f32)] * 6 + [pltpu.VMEM((T, H), f32)],
        compiler_params=pltpu.CompilerParams(
            dimension_semantics=("arbitrary", "arbitrary")),
    )(x, xn, dinv, cnt,
      w0_0, w0_1, w0_2, w1_0, w1_1, w1_2, w2_0, w2_1, w2_2,
      lin1_w, lin1_b.reshape(1, -1), bn_gamma.reshape(1, -1),
      bn_beta.reshape(1, -1), lin2_w, lin2_b.reshape(1, -1))
    return out


# speedup vs baseline: 2.4759x; 2.4759x over previous
import jax
import jax.numpy as jnp
from jax.experimental import pallas as pl
from jax.experimental.pallas import tpu as pltpu

N = 4096
D = 512
H = 128
T1 = 128
NT1 = N // T1
T2 = 512
NT2 = N // T2
THR = 0.9
_BN_SCALE = 1.0 / (1.0 + 1e-5) ** 0.5


def _stats_kernel(x_ref, xn_ref, nrm_ref, dinv_ref, cnt_ref):
    p = pl.program_id(0)
    i = pl.program_id(1)
    rows = pl.ds(i * T1, T1)

    @pl.when(p == 0)
    def _():
        xi = x_ref[...]
        nrm = jnp.maximum(jnp.sqrt(jnp.sum(xi * xi, axis=1, keepdims=True)),
                          1e-12)
        nrm_ref[rows, :] = nrm
        xn_ref[rows, :] = xi / nrm

    @pl.when(p == 1)
    def _():
        xi = xn_ref[rows, :]
        s = jax.lax.dot_general(xi, xn_ref[...], (((1,), (1,)), ((), ())),
                                preferred_element_type=jnp.float32)
        a = jnp.where(s > THR, s, 0.0)
        bm_r = jax.lax.broadcasted_iota(jnp.int32, (N, NT2), 0) // T2
        bm_c = jax.lax.broadcasted_iota(jnp.int32, (N, NT2), 1)
        bm = (bm_r == bm_c).astype(jnp.float32)
        at = jax.lax.dot_general(a, bm, (((1,), (0,)), ((), ())),
                                 preferred_element_type=jnp.float32)
        dsq = jnp.sum(xi * xi, axis=1, keepdims=True)
        ddiag = jnp.where(dsq > THR, dsq, 0.0)
        deg = jnp.sum(at, axis=1, keepdims=True) - ddiag
        dinv_ref[rows, :] = jnp.where(
            deg > 0, jax.lax.rsqrt(jnp.maximum(deg, 1e-12)), 0.0)
        jtile = jax.lax.broadcasted_iota(jnp.int32, (1, NT2), 1)
        diag_here = (jtile == i // (T2 // T1)).astype(jnp.float32)
        cnt_ref[pl.ds(i, 1), :] = (jnp.sum(at, axis=0, keepdims=True)
                                   - diag_here * jnp.sum(ddiag))


def _gcn_kernel(xn_ref, nrm_ref, dinv_ref, cnt_ref,
                w00, w01, w02, w10, w11, w12, w20, w21, w22,
                l1w, l1b, bg, bb, l2w, l2b,
                out_ref,
                v1a_s, v1b_s, v2_s, base_s, h1_s, h2_s, acc_s):
    p = pl.program_id(0)
    i = pl.program_id(1)
    rows = pl.ds(i * T2, T2)
    dv = dinv_ref[rows, :]

    def phase_a(h, wa, wb, wc, v1_s, scale=None):
        b0 = jnp.dot(h, wa[...], preferred_element_type=jnp.float32)
        v1 = jnp.dot(h, wb[...], preferred_element_type=jnp.float32)
        v2 = jnp.dot(h, wc[...], preferred_element_type=jnp.float32)
        if scale is not None:
            b0, v1, v2 = scale * b0, scale * v1, scale * v2
        base_s[rows, :] = b0 - v2
        v1_s[rows, :] = v1
        v2_s[rows, :] = dv * v2

    def spmm_rows(src_s):
        acc_s[...] = jnp.zeros((T2, H), jnp.float32)
        xi = xn_ref[rows, :]

        def body(j, carry):
            @pl.when(cnt_ref[i, j] > 0)
            def _():
                xj = xn_ref[pl.ds(j * T2, T2), :]
                s = jax.lax.dot_general(xi, xj, (((1,), (1,)), ((), ())),
                                        preferred_element_type=jnp.float32)
                colg = jax.lax.broadcasted_iota(jnp.int32, (T2, T2), 1) + j * T2
                rowg = jax.lax.broadcasted_iota(jnp.int32, (T2, T2), 0) + i * T2
                a = jnp.where((s > THR) & (colg != rowg), s, 0.0)
                acc_s[...] += jnp.dot(a, src_s[pl.ds(j * T2, T2), :],
                                      preferred_element_type=jnp.float32)
            return carry

        jax.lax.fori_loop(0, NT2, body, 0)
        return -dv * acc_s[...]

    def phase_b(v1_s):
        m2 = spmm_rows(v2_s)
        v1_s[rows, :] = dv * (v1_s[rows, :] + 2.0 * m2)

    def relu_out(v1_s):
        t = spmm_rows(v1_s)
        return jnp.maximum(base_s[rows, :] + t, 0.0)

    @pl.when(p == 0)
    def _():
        phase_a(xn_ref[rows, :], w00, w01, w02, v1a_s,
                scale=nrm_ref[rows, :])

    @pl.when(p == 1)
    def _():
        phase_b(v1a_s)

    @pl.when(p == 2)
    def _():
        h1 = relu_out(v1a_s)
        h1_s[rows, :] = h1
        phase_a(h1, w10, w11, w12, v1b_s)

    @pl.when(p == 3)
    def _():
        phase_b(v1b_s)

    @pl.when(p == 4)
    def _():
        h2 = relu_out(v1b_s)
        h2_s[rows, :] = h2
        phase_a(h2, w20, w21, w22, v1a_s)

    @pl.when(p == 5)
    def _():
        phase_b(v1a_s)

    @pl.when(p == 6)
    def _():
        h3 = relu_out(v1a_s)
        jk = jnp.concatenate([h1_s[rows, :], h2_s[rows, :], h3], axis=1)
        z = jnp.dot(jk, l1w[...], preferred_element_type=jnp.float32) + l1b[...]
        z = jnp.maximum(z, 0.0)
        z = z * (_BN_SCALE * bg[...]) + bb[...]
        logit = (jnp.dot(z, l2w[...], preferred_element_type=jnp.float32)
                 + l2b[...])
        m = jnp.max(logit, axis=1, keepdims=True)
        e = jnp.exp(logit - m)
        out_ref[rows, :] = e / jnp.sum(e, axis=1, keepdims=True)


def kernel(x, w0_0, w0_1, w0_2, w1_0, w1_1, w1_2, w2_0, w2_1, w2_2,
           lin1_w, lin1_b, bn_gamma, bn_beta, lin2_w, lin2_b):
    f32 = jnp.float32
    xn, nrm, dinv, cntf = pl.pallas_call(
        _stats_kernel,
        grid=(2, NT1),
        in_specs=[pl.BlockSpec((T1, D), lambda p, i: (i, 0))],
        out_specs=[
            pl.BlockSpec((N, D), lambda p, i: (0, 0)),
            pl.BlockSpec((N, 1), lambda p, i: (0, 0)),
            pl.BlockSpec((N, 1), lambda p, i: (0, 0)),
            pl.BlockSpec((NT1, NT2), lambda p, i: (0, 0)),
        ],
        out_shape=[
            jax.ShapeDtypeStruct((N, D), f32),
            jax.ShapeDtypeStruct((N, 1), f32),
            jax.ShapeDtypeStruct((N, 1), f32),
            jax.ShapeDtypeStruct((NT1, NT2), f32),
        ],
        compiler_params=pltpu.CompilerParams(
            dimension_semantics=("arbitrary", "arbitrary")),
    )(x)
    cnt = (cntf.reshape(NT2, NT1 // NT2, NT2).sum(axis=1) > 0.5).astype(
        jnp.int32)

    full = pl.BlockSpec(memory_space=pltpu.VMEM)
    out = pl.pallas_call(
        _gcn_kernel,
        grid=(7, NT2),
        in_specs=[full, full, full,
                  pl.BlockSpec(memory_space=pltpu.SMEM)] + [full] * 15,
        out_specs=pl.BlockSpec((N, 10), lambda p, i: (0, 0)),
        out_shape=jax.ShapeDtypeStruct((N, 10), f32),
        scratch_shapes=[pltpu.VMEM((N, H), f32)] * 6
                       + [pltpu.VMEM((T2, H), f32)],
        compiler_params=pltpu.CompilerParams(
            dimension_semantics=("arbitrary", "arbitrary")),
    )(xn, nrm, dinv, cnt,
      w0_0, w0_1, w0_2, w1_0, w1_1, w1_2, w2_0, w2_1, w2_2,
      lin1_w, lin1_b.reshape(1, -1), bn_gamma.reshape(1, -1),
      bn_beta.reshape(1, -1), lin2_w, lin2_b.reshape(1, -1))
    return out


# speedup vs baseline: 2.5384x; 1.0252x over previous
import jax
import jax.numpy as jnp
from jax.experimental import pallas as pl
from jax.experimental.pallas import tpu as pltpu

N = 4096
D = 512
H = 128
T1 = 128
NT1 = N // T1
T2 = 512
NT2 = N // T2
THR = 0.9
_BN_SCALE = 1.0 / (1.0 + 1e-5) ** 0.5


def _stats_kernel(x_ref, bm_ref, xn_ref, nrm_ref, dinv_ref, cnt_ref, cnt_s):
    p = pl.program_id(0)
    i = pl.program_id(1)
    rows = pl.ds(i * T1, T1)

    @pl.when((p == 0) & (i == 0))
    def _():
        cnt_s[...] = jnp.zeros((NT2, NT2), jnp.float32)

    @pl.when(p == 0)
    def _():
        xi = x_ref[...]
        nrm = jnp.maximum(jnp.sqrt(jnp.sum(xi * xi, axis=1, keepdims=True)),
                          1e-12)
        nrm_ref[rows, :] = nrm
        xn_ref[rows, :] = xi / nrm

    @pl.when(p == 1)
    def _():
        xi = xn_ref[rows, :]
        s = jax.lax.dot_general(xi, xn_ref[...], (((1,), (1,)), ((), ())),
                                preferred_element_type=jnp.float32)
        a = jnp.where(s > THR, s, 0.0)
        at = jax.lax.dot_general(a, bm_ref[...], (((1,), (0,)), ((), ())),
                                 preferred_element_type=jnp.float32)
        dsq = jnp.sum(xi * xi, axis=1, keepdims=True)
        ddiag = jnp.where(dsq > THR, dsq, 0.0)
        deg = jnp.sum(at, axis=1, keepdims=True) - ddiag
        dinv_ref[rows, :] = jnp.where(
            deg > 0, jax.lax.rsqrt(jnp.maximum(deg, 1e-12)), 0.0)
        jtile = jax.lax.broadcasted_iota(jnp.int32, (1, NT2), 1)
        diag_here = (jtile == i // (T2 // T1)).astype(jnp.float32)
        cnt_s[pl.ds(i // (T2 // T1), 1), :] += (
            jnp.sum(at, axis=0, keepdims=True) - diag_here * jnp.sum(ddiag))

    @pl.when((p == 1) & (i == NT1 - 1))
    def _():
        cnt_ref[...] = (cnt_s[...] > 0.5).astype(jnp.int32)


def _gcn_kernel(xn_ref, nrm_ref, dinv_ref, cnt_ref,
                wc0, wc1, wc2,
                l1w, l1b, bg, bb, l2w, l2b,
                out_ref,
                v1a_s, v1b_s, v2_s, base_s, h1_s, h2_s, acc_s):
    p = pl.program_id(0)
    i = pl.program_id(1)
    rows = pl.ds(i * T2, T2)
    dv = dinv_ref[rows, :]

    def phase_a(h, wcat, v1_s, scale=None):
        bvv = jnp.dot(h, wcat[...], preferred_element_type=jnp.float32)
        b0, v1, v2 = bvv[:, :H], bvv[:, H:2 * H], bvv[:, 2 * H:]
        if scale is not None:
            b0, v1, v2 = scale * b0, scale * v1, scale * v2
        base_s[rows, :] = b0 - v2
        v1_s[rows, :] = v1
        v2_s[rows, :] = dv * v2

    def spmm_rows(src_s):
        acc_s[...] = jnp.zeros((T2, H), jnp.float32)
        xi = xn_ref[rows, :]

        def body(j, carry):
            @pl.when(cnt_ref[i, j] > 0)
            def _():
                xj = xn_ref[pl.ds(j * T2, T2), :]
                s = jax.lax.dot_general(xi, xj, (((1,), (1,)), ((), ())),
                                        preferred_element_type=jnp.float32)
                colg = jax.lax.broadcasted_iota(jnp.int32, (T2, T2), 1) + j * T2
                rowg = jax.lax.broadcasted_iota(jnp.int32, (T2, T2), 0) + i * T2
                a = jnp.where((s > THR) & (colg != rowg), s, 0.0)
                acc_s[...] += jnp.dot(a, src_s[pl.ds(j * T2, T2), :],
                                      preferred_element_type=jnp.float32)
            return carry

        jax.lax.fori_loop(0, NT2, body, 0)
        return -dv * acc_s[...]

    def phase_b(v1_s):
        m2 = spmm_rows(v2_s)
        v1_s[rows, :] = dv * (v1_s[rows, :] + 2.0 * m2)

    def relu_out(v1_s):
        t = spmm_rows(v1_s)
        return jnp.maximum(base_s[rows, :] + t, 0.0)

    @pl.when(p == 0)
    def _():
        phase_a(xn_ref[rows, :], wc0, v1a_s, scale=nrm_ref[rows, :])

    @pl.when(p == 1)
    def _():
        phase_b(v1a_s)

    @pl.when(p == 2)
    def _():
        h1 = relu_out(v1a_s)
        h1_s[rows, :] = h1
        phase_a(h1, wc1, v1b_s)

    @pl.when(p == 3)
    def _():
        phase_b(v1b_s)

    @pl.when(p == 4)
    def _():
        h2 = relu_out(v1b_s)
        h2_s[rows, :] = h2
        phase_a(h2, wc2, v1a_s)

    @pl.when(p == 5)
    def _():
        phase_b(v1a_s)

    @pl.when(p == 6)
    def _():
        h3 = relu_out(v1a_s)
        jk = jnp.concatenate([h1_s[rows, :], h2_s[rows, :], h3], axis=1)
        z = jnp.dot(jk, l1w[...], preferred_element_type=jnp.float32) + l1b[...]
        z = jnp.maximum(z, 0.0)
        z = z * (_BN_SCALE * bg[...]) + bb[...]
        logit = (jnp.dot(z, l2w[...], preferred_element_type=jnp.float32)
                 + l2b[...])
        m = jnp.max(logit, axis=1, keepdims=True)
        e = jnp.exp(logit - m)
        out_ref[rows, :] = e / jnp.sum(e, axis=1, keepdims=True)


def kernel(x, w0_0, w0_1, w0_2, w1_0, w1_1, w1_2, w2_0, w2_1, w2_2,
           lin1_w, lin1_b, bn_gamma, bn_beta, lin2_w, lin2_b):
    f32 = jnp.float32
    bm = jnp.repeat(jnp.eye(NT2, dtype=f32), T2, axis=0)
    xn, nrm, dinv, cnt = pl.pallas_call(
        _stats_kernel,
        grid=(2, NT1),
        in_specs=[pl.BlockSpec((T1, D), lambda p, i: (i, 0)),
                  pl.BlockSpec(memory_space=pltpu.VMEM)],
        out_specs=[
            pl.BlockSpec((N, D), lambda p, i: (0, 0)),
            pl.BlockSpec((N, 1), lambda p, i: (0, 0)),
            pl.BlockSpec((N, 1), lambda p, i: (0, 0)),
            pl.BlockSpec((NT2, NT2), lambda p, i: (0, 0)),
        ],
        out_shape=[
            jax.ShapeDtypeStruct((N, D), f32),
            jax.ShapeDtypeStruct((N, 1), f32),
            jax.ShapeDtypeStruct((N, 1), f32),
            jax.ShapeDtypeStruct((NT2, NT2), jnp.int32),
        ],
        scratch_shapes=[pltpu.VMEM((NT2, NT2), f32)],
        compiler_params=pltpu.CompilerParams(
            dimension_semantics=("arbitrary", "arbitrary")),
    )(x, bm)

    wc0 = jnp.concatenate([w0_0, w0_1, w0_2], axis=1)
    wc1 = jnp.concatenate([w1_0, w1_1, w1_2], axis=1)
    wc2 = jnp.concatenate([w2_0, w2_1, w2_2], axis=1)
    full = pl.BlockSpec(memory_space=pltpu.VMEM)
    out = pl.pallas_call(
        _gcn_kernel,
        grid=(7, NT2),
        in_specs=[full, full, full,
                  pl.BlockSpec(memory_space=pltpu.SMEM)] + [full] * 9,
        out_specs=pl.BlockSpec((N, 10), lambda p, i: (0, 0)),
        out_shape=jax.ShapeDtypeStruct((N, 10), f32),
        scratch_shapes=[pltpu.VMEM((N, H), f32)] * 6
                       + [pltpu.VMEM((T2, H), f32)],
        compiler_params=pltpu.CompilerParams(
            dimension_semantics=("arbitrary", "arbitrary")),
    )(xn, nrm, dinv, cnt, wc0, wc1, wc2,
      lin1_w, lin1_b.reshape(1, -1), bn_gamma.reshape(1, -1),
      bn_beta.reshape(1, -1), lin2_w, lin2_b.reshape(1, -1))
    return out


# speedup vs baseline: 3.3398x; 1.3157x over previous
import jax
import jax.numpy as jnp
from jax.experimental import pallas as pl
from jax.experimental.pallas import tpu as pltpu

N = 4096
D = 512
H = 128
T1 = 128
NT1 = N // T1
T2 = 512
NT2 = N // T2
THR = 0.9
_BN_SCALE = 1.0 / (1.0 + 1e-5) ** 0.5


def _stats_kernel(x_ref, bm_ref, nrm_ref, dinv_ref, cnt_ref, xn_s, cnt_s):
    def norm_body(i, c):
        rows = pl.ds(i * T1, T1)
        xi = x_ref[rows, :]
        nrm = jnp.maximum(jnp.sqrt(jnp.sum(xi * xi, axis=1, keepdims=True)),
                          1e-12)
        nrm_ref[rows, :] = nrm
        xn_s[rows, :] = xi / nrm
        return c

    jax.lax.fori_loop(0, NT1, norm_body, 0)
    cnt_s[...] = jnp.zeros((NT2, NT2), jnp.float32)

    def sim_body(i, c):
        rows = pl.ds(i * T1, T1)
        xi = xn_s[rows, :]
        s = jax.lax.dot_general(xi, xn_s[...], (((1,), (1,)), ((), ())),
                                preferred_element_type=jnp.float32)
        a = jnp.where(s > THR, s, 0.0)
        at = jax.lax.dot_general(a, bm_ref[...], (((1,), (0,)), ((), ())),
                                 preferred_element_type=jnp.float32)
        dsq = jnp.sum(xi * xi, axis=1, keepdims=True)
        ddiag = jnp.where(dsq > THR, dsq, 0.0)
        deg = jnp.sum(at, axis=1, keepdims=True) - ddiag
        dinv_ref[rows, :] = jnp.where(
            deg > 0, jax.lax.rsqrt(jnp.maximum(deg, 1e-12)), 0.0)
        jtile = jax.lax.broadcasted_iota(jnp.int32, (1, NT2), 1)
        diag_here = (jtile == i // (T2 // T1)).astype(jnp.float32)
        cnt_s[pl.ds(i // (T2 // T1), 1), :] += (
            jnp.sum(at, axis=0, keepdims=True) - diag_here * jnp.sum(ddiag))
        return c

    jax.lax.fori_loop(0, NT1, sim_body, 0)
    cnt_ref[...] = (cnt_s[...] > 0.5).astype(jnp.int32)


def _gcn_kernel(x_ref, nrm_ref, dinv_ref, cnt_ref,
                wc0, wc1, wc2, l1w, l1b, bg, bb, l2w, l2b,
                out_ref,
                v1a_s, v1b_s, v2_s, base_s, h1_s, h2_s, h3_s, acc_s):

    def dv_of(i):
        return dinv_ref[pl.ds(i * T2, T2), :]

    def phase_a(i, h, wcat, v1_s):
        rows = pl.ds(i * T2, T2)
        bvv = jnp.dot(h, wcat[...], preferred_element_type=jnp.float32)
        b0, v1, v2 = bvv[:, :H], bvv[:, H:2 * H], bvv[:, 2 * H:]
        base_s[rows, :] = b0 - v2
        v1_s[rows, :] = v1
        v2_s[rows, :] = dv_of(i) * v2

    def row_active(i):
        def rbody(j, r):
            return jnp.maximum(r, cnt_ref[i, j])
        return jax.lax.fori_loop(0, NT2, rbody, jnp.int32(0))

    def spmm_rows(i, src_s):
        acc_s[...] = jnp.zeros((T2, H), jnp.float32)
        rows = pl.ds(i * T2, T2)
        xi = x_ref[rows, :] / nrm_ref[rows, :]

        def body(j, carry):
            @pl.when(cnt_ref[i, j] > 0)
            def _():
                cols = pl.ds(j * T2, T2)
                xj = x_ref[cols, :] / nrm_ref[cols, :]
                s = jax.lax.dot_general(xi, xj, (((1,), (1,)), ((), ())),
                                        preferred_element_type=jnp.float32)
                colg = jax.lax.broadcasted_iota(jnp.int32, (T2, T2), 1) + j * T2
                rowg = jax.lax.broadcasted_iota(jnp.int32, (T2, T2), 0) + i * T2
                a = jnp.where((s > THR) & (colg != rowg), s, 0.0)
                acc_s[...] += jnp.dot(a, src_s[cols, :],
                                      preferred_element_type=jnp.float32)
            return carry

        jax.lax.fori_loop(0, NT2, body, 0)
        return -dv_of(i) * acc_s[...]

    def phase_b(i, v1_s):
        rows = pl.ds(i * T2, T2)
        act = row_active(i)

        @pl.when(act > 0)
        def _():
            m2 = spmm_rows(i, v2_s)
            v1_s[rows, :] = dv_of(i) * (v1_s[rows, :] + 2.0 * m2)

        @pl.when(act == 0)
        def _():
            v1_s[rows, :] = dv_of(i) * v1_s[rows, :]

    def relu_into(i, v1_s, dst_s):
        rows = pl.ds(i * T2, T2)
        act = row_active(i)

        @pl.when(act > 0)
        def _():
            dst_s[rows, :] = jnp.maximum(
                base_s[rows, :] + spmm_rows(i, v1_s), 0.0)

        @pl.when(act == 0)
        def _():
            dst_s[rows, :] = jnp.maximum(base_s[rows, :], 0.0)

    def head(i):
        rows = pl.ds(i * T2, T2)
        jk = jnp.concatenate(
            [h1_s[rows, :], h2_s[rows, :], h3_s[rows, :]], axis=1)
        z = jnp.dot(jk, l1w[...], preferred_element_type=jnp.float32) + l1b[...]
        z = jnp.maximum(z, 0.0)
        z = z * (_BN_SCALE * bg[...]) + bb[...]
        logit = (jnp.dot(z, l2w[...], preferred_element_type=jnp.float32)
                 + l2b[...])
        m = jnp.max(logit, axis=1, keepdims=True)
        e = jnp.exp(logit - m)
        out_ref[rows, :] = e / jnp.sum(e, axis=1, keepdims=True)

    def loop(fn):
        jax.lax.fori_loop(0, NT2, lambda i, c: (fn(i), c)[1], 0)

    loop(lambda i: phase_a(i, x_ref[pl.ds(i * T2, T2), :], wc0, v1a_s))
    loop(lambda i: phase_b(i, v1a_s))

    def c0a1(i):
        relu_into(i, v1a_s, h1_s)
        phase_a(i, h1_s[pl.ds(i * T2, T2), :], wc1, v1b_s)

    loop(c0a1)
    loop(lambda i: phase_b(i, v1b_s))

    def c1a2(i):
        relu_into(i, v1b_s, h2_s)
        phase_a(i, h2_s[pl.ds(i * T2, T2), :], wc2, v1a_s)

    loop(c1a2)
    loop(lambda i: phase_b(i, v1a_s))

    def c2head(i):
        relu_into(i, v1a_s, h3_s)
        head(i)

    loop(c2head)


def kernel(x, w0_0, w0_1, w0_2, w1_0, w1_1, w1_2, w2_0, w2_1, w2_2,
           lin1_w, lin1_b, bn_gamma, bn_beta, lin2_w, lin2_b):
    f32 = jnp.float32
    bm = jnp.repeat(jnp.eye(NT2, dtype=f32), T2, axis=0)
    full = pl.BlockSpec(memory_space=pltpu.VMEM)
    nrm, dinv, cnt = pl.pallas_call(
        _stats_kernel,
        in_specs=[full, full],
        out_specs=[full, full, full],
        out_shape=[
            jax.ShapeDtypeStruct((N, 1), f32),
            jax.ShapeDtypeStruct((N, 1), f32),
            jax.ShapeDtypeStruct((NT2, NT2), jnp.int32),
        ],
        scratch_shapes=[pltpu.VMEM((N, D), f32), pltpu.VMEM((NT2, NT2), f32)],
    )(x, bm)

    wc0 = jnp.concatenate([w0_0, w0_1, w0_2], axis=1)
    wc1 = jnp.concatenate([w1_0, w1_1, w1_2], axis=1)
    wc2 = jnp.concatenate([w2_0, w2_1, w2_2], axis=1)
    out = pl.pallas_call(
        _gcn_kernel,
        in_specs=[full, full, full,
                  pl.BlockSpec(memory_space=pltpu.SMEM)] + [full] * 9,
        out_specs=full,
        out_shape=jax.ShapeDtypeStruct((N, 10), f32),
        scratch_shapes=[pltpu.VMEM((N, H), f32)] * 7
                       + [pltpu.VMEM((T2, H), f32)],
    )(x, nrm, dinv, cnt, wc0, wc1, wc2,
      lin1_w, lin1_b.reshape(1, -1), bn_gamma.reshape(1, -1),
      bn_beta.reshape(1, -1), lin2_w, lin2_b.reshape(1, -1))
    return out


# speedup vs baseline: 3.4232x; 1.0250x over previous
import jax
import jax.numpy as jnp
from jax.experimental import pallas as pl
from jax.experimental.pallas import tpu as pltpu

N = 4096
D = 512
H = 128
T1 = 128
NT1 = N // T1
T2 = 512
NT2 = N // T2
THR = 0.9
SUS_THR = 0.88
_BN_SCALE = 1.0 / (1.0 + 1e-5) ** 0.5


def _stats_kernel(x_ref, bm_ref, nrm_ref, cnt_ref, xnb_s, cnt_s):
    def norm_body(i, c):
        rows = pl.ds(i * T1, T1)
        xi = x_ref[rows, :]
        nrm = jnp.maximum(jnp.sqrt(jnp.sum(xi * xi, axis=1, keepdims=True)),
                          1e-12)
        nrm_ref[rows, :] = nrm
        xnb_s[rows, :] = (xi / nrm).astype(jnp.bfloat16)
        return c

    jax.lax.fori_loop(0, NT1, norm_body, 0)
    cnt_s[...] = jnp.zeros((NT2, NT2), jnp.float32)

    def sim_body(i, c):
        rows = pl.ds(i * T1, T1)
        xib = xnb_s[rows, :]
        s = jax.lax.dot_general(xib, xnb_s[...], (((1,), (1,)), ((), ())),
                                preferred_element_type=jnp.float32)
        colg = jax.lax.broadcasted_iota(jnp.int32, (T1, N), 1)
        rowg = jax.lax.broadcasted_iota(jnp.int32, (T1, N), 0) + i * T1
        sus = ((s > SUS_THR) & (colg != rowg)).astype(jnp.float32)
        at = jax.lax.dot_general(sus, bm_ref[...], (((1,), (0,)), ((), ())),
                                 preferred_element_type=jnp.float32)
        cnt_s[pl.ds(i // (T2 // T1), 1), :] += jnp.sum(at, axis=0,
                                                       keepdims=True)
        return c

    jax.lax.fori_loop(0, NT1, sim_body, 0)
    cnt_ref[...] = (cnt_s[...] > 0.5).astype(jnp.int32)


def _gcn_kernel(x_ref, nrm_ref, cnt_ref,
                wc0, wc1, wc2, l1w, l1b, bg, bb, l2w, l2b,
                out_ref,
                v1a_s, v1b_s, v2_s, base_s, h1_s, h2_s, h3_s, dinv_s, acc_s):

    def dv_of(i):
        return dinv_s[pl.ds(i * T2, T2), :]

    def row_active(i):
        def rbody(j, r):
            return jnp.maximum(r, cnt_ref[i, j])
        return jax.lax.fori_loop(0, NT2, rbody, jnp.int32(0))

    def exact_tile(i, j):
        rows = pl.ds(i * T2, T2)
        cols = pl.ds(j * T2, T2)
        xi = x_ref[rows, :] / nrm_ref[rows, :]
        xj = x_ref[cols, :] / nrm_ref[cols, :]
        s = jax.lax.dot_general(xi, xj, (((1,), (1,)), ((), ())),
                                preferred_element_type=jnp.float32)
        colg = jax.lax.broadcasted_iota(jnp.int32, (T2, T2), 1) + j * T2
        rowg = jax.lax.broadcasted_iota(jnp.int32, (T2, T2), 0) + i * T2
        return jnp.where((s > THR) & (colg != rowg), s, 0.0)

    def deg_dinv(i):
        rows = pl.ds(i * T2, T2)
        act = row_active(i)

        @pl.when(act > 0)
        def _():
            acc_s[...] = jnp.zeros((T2, H), jnp.float32)

            def dbody(j, carry):
                @pl.when(cnt_ref[i, j] > 0)
                def _():
                    a = exact_tile(i, j)
                    acc_s[:, 0:1] += jnp.sum(a, axis=1, keepdims=True)
                return carry

            jax.lax.fori_loop(0, NT2, dbody, 0)
            deg = acc_s[:, 0:1]
            dinv_s[rows, :] = jnp.where(
                deg > 0, jax.lax.rsqrt(jnp.maximum(deg, 1e-12)), 0.0)

        @pl.when(act == 0)
        def _():
            dinv_s[rows, :] = jnp.zeros((T2, 1), jnp.float32)

    def phase_a(i, h, wcat, v1_s):
        rows = pl.ds(i * T2, T2)
        bvv = jnp.dot(h, wcat[...], preferred_element_type=jnp.float32)
        b0, v1, v2 = bvv[:, :H], bvv[:, H:2 * H], bvv[:, 2 * H:]
        base_s[rows, :] = b0 - v2
        v1_s[rows, :] = v1
        v2_s[rows, :] = dv_of(i) * v2

    def spmm_rows(i, src_s):
        acc_s[...] = jnp.zeros((T2, H), jnp.float32)

        def body(j, carry):
            @pl.when(cnt_ref[i, j] > 0)
            def _():
                a = exact_tile(i, j)
                acc_s[...] += jnp.dot(a, src_s[pl.ds(j * T2, T2), :],
                                      preferred_element_type=jnp.float32)
            return carry

        jax.lax.fori_loop(0, NT2, body, 0)
        return -dv_of(i) * acc_s[...]

    def phase_b(i, v1_s):
        rows = pl.ds(i * T2, T2)
        act = row_active(i)

        @pl.when(act > 0)
        def _():
            m2 = spmm_rows(i, v2_s)
            v1_s[rows, :] = dv_of(i) * (v1_s[rows, :] + 2.0 * m2)

        @pl.when(act == 0)
        def _():
            v1_s[rows, :] = dv_of(i) * v1_s[rows, :]

    def relu_into(i, v1_s, dst_s):
        rows = pl.ds(i * T2, T2)
        act = row_active(i)

        @pl.when(act > 0)
        def _():
            dst_s[rows, :] = jnp.maximum(
                base_s[rows, :] + spmm_rows(i, v1_s), 0.0)

        @pl.when(act == 0)
        def _():
            dst_s[rows, :] = jnp.maximum(base_s[rows, :], 0.0)

    def head(i):
        rows = pl.ds(i * T2, T2)
        jk = jnp.concatenate(
            [h1_s[rows, :], h2_s[rows, :], h3_s[rows, :]], axis=1)
        z = jnp.dot(jk, l1w[...], preferred_element_type=jnp.float32) + l1b[...]
        z = jnp.maximum(z, 0.0)
        z = z * (_BN_SCALE * bg[...]) + bb[...]
        logit = (jnp.dot(z, l2w[...], preferred_element_type=jnp.float32)
                 + l2b[...])
        m = jnp.max(logit, axis=1, keepdims=True)
        e = jnp.exp(logit - m)
        out_ref[rows, :] = e / jnp.sum(e, axis=1, keepdims=True)

    def loop(fn):
        jax.lax.fori_loop(0, NT2, lambda i, c: (fn(i), c)[1], 0)

    def dega0(i):
        deg_dinv(i)
        phase_a(i, x_ref[pl.ds(i * T2, T2), :], wc0, v1a_s)

    loop(dega0)
    loop(lambda i: phase_b(i, v1a_s))

    def c0a1(i):
        relu_into(i, v1a_s, h1_s)
        phase_a(i, h1_s[pl.ds(i * T2, T2), :], wc1, v1b_s)

    loop(c0a1)
    loop(lambda i: phase_b(i, v1b_s))

    def c1a2(i):
        relu_into(i, v1b_s, h2_s)
        phase_a(i, h2_s[pl.ds(i * T2, T2), :], wc2, v1a_s)

    loop(c1a2)
    loop(lambda i: phase_b(i, v1a_s))

    def c2head(i):
        relu_into(i, v1a_s, h3_s)
        head(i)

    loop(c2head)


def kernel(x, w0_0, w0_1, w0_2, w1_0, w1_1, w1_2, w2_0, w2_1, w2_2,
           lin1_w, lin1_b, bn_gamma, bn_beta, lin2_w, lin2_b):
    f32 = jnp.float32
    bm = jnp.repeat(jnp.eye(NT2, dtype=f32), T2, axis=0)
    full = pl.BlockSpec(memory_space=pltpu.VMEM)
    nrm, cnt = pl.pallas_call(
        _stats_kernel,
        in_specs=[full, full],
        out_specs=[full, full],
        out_shape=[
            jax.ShapeDtypeStruct((N, 1), f32),
            jax.ShapeDtypeStruct((NT2, NT2), jnp.int32),
        ],
        scratch_shapes=[pltpu.VMEM((N, D), jnp.bfloat16),
                        pltpu.VMEM((NT2, NT2), f32)],
    )(x, bm)

    wc0 = jnp.concatenate([w0_0, w0_1, w0_2], axis=1)
    wc1 = jnp.concatenate([w1_0, w1_1, w1_2], axis=1)
    wc2 = jnp.concatenate([w2_0, w2_1, w2_2], axis=1)
    out = pl.pallas_call(
        _gcn_kernel,
        in_specs=[full, full,
                  pl.BlockSpec(memory_space=pltpu.SMEM)] + [full] * 9,
        out_specs=full,
        out_shape=jax.ShapeDtypeStruct((N, 10), f32),
        scratch_shapes=[pltpu.VMEM((N, H), f32)] * 7
                       + [pltpu.VMEM((N, 1), f32)]
                       + [pltpu.VMEM((T2, H), f32)],
    )(x, nrm, cnt, wc0, wc1, wc2,
      lin1_w, lin1_b.reshape(1, -1), bn_gamma.reshape(1, -1),
      bn_beta.reshape(1, -1), lin2_w, lin2_b.reshape(1, -1))
    return out


# speedup vs baseline: 4.7054x; 1.3746x over previous
import jax
import jax.numpy as jnp
from jax.experimental import pallas as pl
from jax.experimental.pallas import tpu as pltpu

N = 4096
D = 512
H = 128
T1 = 128
NT1 = N // T1
T2 = 512
NT2 = N // T2
THR = 0.9
SUS_THR = 0.88
_BN_SCALE = 1.0 / (1.0 + 1e-5) ** 0.5


def _stats_kernel(x_ref, nrm_ref, cnt_ref, xnb_s, cnt_s):
    def norm_body(i, c):
        rows = pl.ds(i * T1, T1)
        xi = x_ref[rows, :]
        nrm = jnp.maximum(jnp.sqrt(jnp.sum(xi * xi, axis=1, keepdims=True)),
                          1e-12)
        nrm_ref[rows, :] = nrm
        xnb_s[rows, :] = (xi / nrm).astype(jnp.bfloat16)
        return c

    jax.lax.fori_loop(0, NT1, norm_body, 0)
    cnt_s[...] = jnp.zeros((NT2 * NT2, 1), jnp.float32)

    def pair_body(k, c):
        i = k // NT2
        j = k % NT2

        @pl.when(j >= i)
        def _():
            xib = xnb_s[pl.ds(i * T2, T2), :]
            xjb = xnb_s[pl.ds(j * T2, T2), :]
            s = jax.lax.dot_general(xib, xjb, (((1,), (1,)), ((), ())),
                                    preferred_element_type=jnp.float32)
            colg = jax.lax.broadcasted_iota(jnp.int32, (T2, T2), 1)
            rowg = jax.lax.broadcasted_iota(jnp.int32, (T2, T2), 0)
            off_diag = (colg > rowg) | (j != i)
            sus = ((s > SUS_THR) & off_diag).astype(jnp.float32)
            v = jnp.sum(sus, keepdims=True)
            cnt_s[pl.ds(i * NT2 + j, 1), :] += v
            cnt_s[pl.ds(j * NT2 + i, 1), :] += v
        return c

    jax.lax.fori_loop(0, NT2 * NT2, pair_body, 0)
    cnt_ref[...] = (cnt_s[...] > 0.5).astype(jnp.int32)


def _gcn_kernel(x_ref, nrm_ref, cnt_ref,
                wc0, wc1, wc2, l1w, l1b, bg, bb, l2w, l2b,
                out_ref,
                v1a_s, v1b_s, v2_s, base_s, h1_s, h2_s, h3_s, dinv_s, acc_s):

    def dv_of(i):
        return dinv_s[pl.ds(i * T2, T2), :]

    def row_active(i):
        def rbody(j, r):
            return jnp.maximum(r, cnt_ref[i, j])
        return jax.lax.fori_loop(0, NT2, rbody, jnp.int32(0))

    def exact_tile(i, j):
        rows = pl.ds(i * T2, T2)
        cols = pl.ds(j * T2, T2)
        xi = x_ref[rows, :] / nrm_ref[rows, :]
        xj = x_ref[cols, :] / nrm_ref[cols, :]
        s = jax.lax.dot_general(xi, xj, (((1,), (1,)), ((), ())),
                                preferred_element_type=jnp.float32)
        colg = jax.lax.broadcasted_iota(jnp.int32, (T2, T2), 1) + j * T2
        rowg = jax.lax.broadcasted_iota(jnp.int32, (T2, T2), 0) + i * T2
        return jnp.where((s > THR) & (colg != rowg), s, 0.0)

    def deg_dinv(i):
        rows = pl.ds(i * T2, T2)
        act = row_active(i)

        @pl.when(act > 0)
        def _():
            acc_s[...] = jnp.zeros((T2, H), jnp.float32)

            def dbody(j, carry):
                @pl.when(cnt_ref[i, j] > 0)
                def _():
                    a = exact_tile(i, j)
                    acc_s[:, 0:1] += jnp.sum(a, axis=1, keepdims=True)
                return carry

            jax.lax.fori_loop(0, NT2, dbody, 0)
            deg = acc_s[:, 0:1]
            dinv_s[rows, :] = jnp.where(
                deg > 0, jax.lax.rsqrt(jnp.maximum(deg, 1e-12)), 0.0)

        @pl.when(act == 0)
        def _():
            dinv_s[rows, :] = jnp.zeros((T2, 1), jnp.float32)

    def phase_a(i, h, wcat, v1_s):
        rows = pl.ds(i * T2, T2)
        bvv = jnp.dot(h, wcat[...], preferred_element_type=jnp.float32)
        b0, v1, v2 = bvv[:, :H], bvv[:, H:2 * H], bvv[:, 2 * H:]
        base_s[rows, :] = b0 - v2
        v1_s[rows, :] = v1
        v2_s[rows, :] = dv_of(i) * v2

    def spmm_rows(i, src_s):
        acc_s[...] = jnp.zeros((T2, H), jnp.float32)

        def body(j, carry):
            @pl.when(cnt_ref[i, j] > 0)
            def _():
                a = exact_tile(i, j)
                acc_s[...] += jnp.dot(a, src_s[pl.ds(j * T2, T2), :],
                                      preferred_element_type=jnp.float32)
            return carry

        jax.lax.fori_loop(0, NT2, body, 0)
        return -dv_of(i) * acc_s[...]

    def phase_b(i, v1_s):
        rows = pl.ds(i * T2, T2)
        act = row_active(i)

        @pl.when(act > 0)
        def _():
            m2 = spmm_rows(i, v2_s)
            v1_s[rows, :] = dv_of(i) * (v1_s[rows, :] + 2.0 * m2)

        @pl.when(act == 0)
        def _():
            v1_s[rows, :] = dv_of(i) * v1_s[rows, :]

    def relu_into(i, v1_s, dst_s):
        rows = pl.ds(i * T2, T2)
        act = row_active(i)

        @pl.when(act > 0)
        def _():
            dst_s[rows, :] = jnp.maximum(
                base_s[rows, :] + spmm_rows(i, v1_s), 0.0)

        @pl.when(act == 0)
        def _():
            dst_s[rows, :] = jnp.maximum(base_s[rows, :], 0.0)

    def head(i):
        rows = pl.ds(i * T2, T2)
        jk = jnp.concatenate(
            [h1_s[rows, :], h2_s[rows, :], h3_s[rows, :]], axis=1)
        z = jnp.dot(jk, l1w[...], preferred_element_type=jnp.float32) + l1b[...]
        z = jnp.maximum(z, 0.0)
        z = z * (_BN_SCALE * bg[...]) + bb[...]
        logit = (jnp.dot(z, l2w[...], preferred_element_type=jnp.float32)
                 + l2b[...])
        m = jnp.max(logit, axis=1, keepdims=True)
        e = jnp.exp(logit - m)
        out_ref[rows, :] = e / jnp.sum(e, axis=1, keepdims=True)

    def loop(fn):
        jax.lax.fori_loop(0, NT2, lambda i, c: (fn(i), c)[1], 0)

    def dega0(i):
        deg_dinv(i)
        phase_a(i, x_ref[pl.ds(i * T2, T2), :], wc0, v1a_s)

    loop(dega0)
    loop(lambda i: phase_b(i, v1a_s))

    def c0a1(i):
        relu_into(i, v1a_s, h1_s)
        phase_a(i, h1_s[pl.ds(i * T2, T2), :], wc1, v1b_s)

    loop(c0a1)
    loop(lambda i: phase_b(i, v1b_s))

    def c1a2(i):
        relu_into(i, v1b_s, h2_s)
        phase_a(i, h2_s[pl.ds(i * T2, T2), :], wc2, v1a_s)

    loop(c1a2)
    loop(lambda i: phase_b(i, v1a_s))

    def c2head(i):
        relu_into(i, v1a_s, h3_s)
        head(i)

    loop(c2head)


def kernel(x, w0_0, w0_1, w0_2, w1_0, w1_1, w1_2, w2_0, w2_1, w2_2,
           lin1_w, lin1_b, bn_gamma, bn_beta, lin2_w, lin2_b):
    f32 = jnp.float32
    full = pl.BlockSpec(memory_space=pltpu.VMEM)
    nrm, cnt = pl.pallas_call(
        _stats_kernel,
        in_specs=[full],
        out_specs=[full, full],
        out_shape=[
            jax.ShapeDtypeStruct((N, 1), f32),
            jax.ShapeDtypeStruct((NT2 * NT2, 1), jnp.int32),
        ],
        scratch_shapes=[pltpu.VMEM((N, D), jnp.bfloat16),
                        pltpu.VMEM((NT2 * NT2, 1), f32)],
    )(x)
    cnt = cnt.reshape(NT2, NT2)

    wc0 = jnp.concatenate([w0_0, w0_1, w0_2], axis=1)
    wc1 = jnp.concatenate([w1_0, w1_1, w1_2], axis=1)
    wc2 = jnp.concatenate([w2_0, w2_1, w2_2], axis=1)
    out = pl.pallas_call(
        _gcn_kernel,
        in_specs=[full, full,
                  pl.BlockSpec(memory_space=pltpu.SMEM)] + [full] * 9,
        out_specs=full,
        out_shape=jax.ShapeDtypeStruct((N, 10), f32),
        scratch_shapes=[pltpu.VMEM((N, H), f32)] * 7
                       + [pltpu.VMEM((N, 1), f32)]
                       + [pltpu.VMEM((T2, H), f32)],
    )(x, nrm, cnt, wc0, wc1, wc2,
      lin1_w, lin1_b.reshape(1, -1), bn_gamma.reshape(1, -1),
      bn_beta.reshape(1, -1), lin2_w, lin2_b.reshape(1, -1))
    return out


# speedup vs baseline: 4.9258x; 1.0468x over previous
import jax
import jax.numpy as jnp
from jax.experimental import pallas as pl
from jax.experimental.pallas import tpu as pltpu

N = 4096
D = 512
H = 128
T1 = 128
NT1 = N // T1
T2 = 512
NT2 = N // T2
THR = 0.9
SUS_THR = 0.88
_BN_SCALE = 1.0 / (1.0 + 1e-5) ** 0.5


def _stats_kernel(x_ref, nrm_ref, cnt_ref, xnb_s, cnt_s):
    def norm_body(i, c):
        rows = pl.ds(i * T2, T2)
        xi = x_ref[rows, :]
        nrm = jnp.maximum(jnp.sqrt(jnp.sum(xi * xi, axis=1, keepdims=True)),
                          1e-12)
        nrm_ref[rows, :] = nrm
        xnb_s[rows, :] = (xi / nrm).astype(jnp.bfloat16)
        return c

    jax.lax.fori_loop(0, NT2, norm_body, 0)
    cnt_s[...] = jnp.zeros((NT2 * NT2, 1), jnp.float32)

    def pair_body(k, c):
        i = k // NT2
        j = k % NT2

        @pl.when(j >= i)
        def _():
            xib = xnb_s[pl.ds(i * T2, T2), :]
            xjb = xnb_s[pl.ds(j * T2, T2), :]
            s = jax.lax.dot_general(xib, xjb, (((1,), (1,)), ((), ())),
                                    preferred_element_type=jnp.float32)
            colg = jax.lax.broadcasted_iota(jnp.int32, (T2, T2), 1)
            rowg = jax.lax.broadcasted_iota(jnp.int32, (T2, T2), 0)
            off_diag = (colg > rowg) | (j != i)
            sus = ((s > SUS_THR) & off_diag).astype(jnp.float32)
            v = jnp.sum(sus, keepdims=True)
            cnt_s[pl.ds(i * NT2 + j, 1), :] += v
            cnt_s[pl.ds(j * NT2 + i, 1), :] += v
        return c

    jax.lax.fori_loop(0, NT2 * NT2, pair_body, 0)
    cnt_ref[...] = (cnt_s[...] > 0.5).astype(jnp.int32)


def _gcn_kernel(x_ref, nrm_ref, cnt_ref,
                wc0, wc1, wc2, l1w, l1b, bg, bb, l2w, l2b,
                out_ref,
                v1a_s, v1b_s, v2_s, base_s, h1_s, h2_s, h3_s, dinv_s, acc_s):

    def dv_of(i):
        return dinv_s[pl.ds(i * T2, T2), :]

    def row_active(i):
        def rbody(j, r):
            return jnp.maximum(r, cnt_ref[i, j])
        return jax.lax.fori_loop(0, NT2, rbody, jnp.int32(0))

    def exact_tile(i, j):
        rows = pl.ds(i * T2, T2)
        cols = pl.ds(j * T2, T2)
        xi = x_ref[rows, :] / nrm_ref[rows, :]
        xj = x_ref[cols, :] / nrm_ref[cols, :]
        s = jax.lax.dot_general(xi, xj, (((1,), (1,)), ((), ())),
                                preferred_element_type=jnp.float32)
        colg = jax.lax.broadcasted_iota(jnp.int32, (T2, T2), 1) + j * T2
        rowg = jax.lax.broadcasted_iota(jnp.int32, (T2, T2), 0) + i * T2
        return jnp.where((s > THR) & (colg != rowg), s, 0.0)

    def deg_dinv(i):
        rows = pl.ds(i * T2, T2)
        act = row_active(i)

        @pl.when(act > 0)
        def _():
            acc_s[...] = jnp.zeros((T2, H), jnp.float32)

            def dbody(j, carry):
                @pl.when(cnt_ref[i, j] > 0)
                def _():
                    a = exact_tile(i, j)
                    acc_s[:, 0:1] += jnp.sum(a, axis=1, keepdims=True)
                return carry

            jax.lax.fori_loop(0, NT2, dbody, 0)
            deg = acc_s[:, 0:1]
            dinv_s[rows, :] = jnp.where(
                deg > 0, jax.lax.rsqrt(jnp.maximum(deg, 1e-12)), 0.0)

        @pl.when(act == 0)
        def _():
            dinv_s[rows, :] = jnp.zeros((T2, 1), jnp.float32)

    def phase_a(i, h, wcat, v1_s):
        rows = pl.ds(i * T2, T2)
        bvv = jnp.dot(h, wcat[...], preferred_element_type=jnp.float32)
        b0, v1, v2 = bvv[:, :H], bvv[:, H:2 * H], bvv[:, 2 * H:]
        base_s[rows, :] = b0 - v2

        @pl.when(row_active(i) > 0)
        def _():
            v1_s[rows, :] = v1
            v2_s[rows, :] = dv_of(i) * v2

    def spmm_rows(i, src_s):
        acc_s[...] = jnp.zeros((T2, H), jnp.float32)

        def body(j, carry):
            @pl.when(cnt_ref[i, j] > 0)
            def _():
                a = exact_tile(i, j)
                acc_s[...] += jnp.dot(a, src_s[pl.ds(j * T2, T2), :],
                                      preferred_element_type=jnp.float32)
            return carry

        jax.lax.fori_loop(0, NT2, body, 0)
        return -dv_of(i) * acc_s[...]

    def phase_b(i, v1_s):
        rows = pl.ds(i * T2, T2)
        act = row_active(i)

        @pl.when(act > 0)
        def _():
            m2 = spmm_rows(i, v2_s)
            v1_s[rows, :] = dv_of(i) * (v1_s[rows, :] + 2.0 * m2)

    def relu_into(i, v1_s, dst_s):
        rows = pl.ds(i * T2, T2)
        act = row_active(i)

        @pl.when(act > 0)
        def _():
            dst_s[rows, :] = jnp.maximum(
                base_s[rows, :] + spmm_rows(i, v1_s), 0.0)

        @pl.when(act == 0)
        def _():
            dst_s[rows, :] = jnp.maximum(base_s[rows, :], 0.0)

    def head(i):
        rows = pl.ds(i * T2, T2)
        z = (jnp.dot(h1_s[rows, :], l1w[:H, :],
                     preferred_element_type=jnp.float32)
             + jnp.dot(h2_s[rows, :], l1w[H:2 * H, :],
                       preferred_element_type=jnp.float32)
             + jnp.dot(h3_s[rows, :], l1w[2 * H:, :],
                       preferred_element_type=jnp.float32)
             + l1b[...])
        z = jnp.maximum(z, 0.0)
        z = z * (_BN_SCALE * bg[...]) + bb[...]
        logit = (jnp.dot(z, l2w[...], preferred_element_type=jnp.float32)
                 + l2b[...])
        m = jnp.max(logit, axis=1, keepdims=True)
        e = jnp.exp(logit - m)
        out_ref[rows, :] = e / jnp.sum(e, axis=1, keepdims=True)

    def loop(fn):
        jax.lax.fori_loop(0, NT2, lambda i, c: (fn(i), c)[1], 0)

    def dega0(i):
        deg_dinv(i)
        phase_a(i, x_ref[pl.ds(i * T2, T2), :], wc0, v1a_s)

    loop(dega0)
    loop(lambda i: phase_b(i, v1a_s))

    def c0a1(i):
        relu_into(i, v1a_s, h1_s)
        phase_a(i, h1_s[pl.ds(i * T2, T2), :], wc1, v1b_s)

    loop(c0a1)
    loop(lambda i: phase_b(i, v1b_s))

    def c1a2(i):
        relu_into(i, v1b_s, h2_s)
        phase_a(i, h2_s[pl.ds(i * T2, T2), :], wc2, v1a_s)

    loop(c1a2)
    loop(lambda i: phase_b(i, v1a_s))

    def c2head(i):
        relu_into(i, v1a_s, h3_s)
        head(i)

    loop(c2head)


def kernel(x, w0_0, w0_1, w0_2, w1_0, w1_1, w1_2, w2_0, w2_1, w2_2,
           lin1_w, lin1_b, bn_gamma, bn_beta, lin2_w, lin2_b):
    f32 = jnp.float32
    full = pl.BlockSpec(memory_space=pltpu.VMEM)
    nrm, cnt = pl.pallas_call(
        _stats_kernel,
        in_specs=[full],
        out_specs=[full, full],
        out_shape=[
            jax.ShapeDtypeStruct((N, 1), f32),
            jax.ShapeDtypeStruct((NT2 * NT2, 1), jnp.int32),
        ],
        scratch_shapes=[pltpu.VMEM((N, D), jnp.bfloat16),
                        pltpu.VMEM((NT2 * NT2, 1), f32)],
    )(x)
    cnt = cnt.reshape(NT2, NT2)

    wc0 = jnp.concatenate([w0_0, w0_1, w0_2], axis=1)
    wc1 = jnp.concatenate([w1_0, w1_1, w1_2], axis=1)
    wc2 = jnp.concatenate([w2_0, w2_1, w2_2], axis=1)
    out = pl.pallas_call(
        _gcn_kernel,
        in_specs=[full, full,
                  pl.BlockSpec(memory_space=pltpu.SMEM)] + [full] * 9,
        out_specs=full,
        out_shape=jax.ShapeDtypeStruct((N, 10), f32),
        scratch_shapes=[pltpu.VMEM((N, H), f32)] * 7
                       + [pltpu.VMEM((N, 1), f32)]
                       + [pltpu.VMEM((T2, H), f32)],
    )(x, nrm, cnt, wc0, wc1, wc2,
      lin1_w, lin1_b.reshape(1, -1), bn_gamma.reshape(1, -1),
      bn_beta.reshape(1, -1), lin2_w, lin2_b.reshape(1, -1))
    return out


# speedup vs baseline: 5.5511x; 1.1269x over previous
import jax
import jax.numpy as jnp
from jax.experimental import pallas as pl
from jax.experimental.pallas import tpu as pltpu

N = 4096
D = 512
H = 128
T1 = 128
NT1 = N // T1
T2 = 512
NT2 = N // T2
THR = 0.9
SUS_THR = 0.88
_BN_SCALE = 1.0 / (1.0 + 1e-5) ** 0.5


def _gcn_kernel(x_ref,
                wc0, wc1, wc2, l1w, l1b, bg, bb, l2w, l2b,
                out_ref,
                v1a_s, v1b_s, v2_s, base_s, h1_s, h2_s, h3_s, dinv_s, acc_s,
                nrm_s, xnb_s, cnt_sm):
    nrm_ref = nrm_s

    def norm_body(i, c):
        rows = pl.ds(i * T2, T2)
        xi = x_ref[rows, :]
        nrm = jnp.maximum(jnp.sqrt(jnp.sum(xi * xi, axis=1, keepdims=True)),
                          1e-12)
        nrm_s[rows, :] = nrm
        xnb_s[rows, :] = (xi / nrm).astype(jnp.bfloat16)
        return c

    jax.lax.fori_loop(0, NT2, norm_body, 0)

    def zero_body(k, c):
        cnt_sm[k] = 0.0
        return c

    jax.lax.fori_loop(0, NT2 * NT2, zero_body, 0)

    def pair_body(k, c):
        i = k // NT2
        j = k % NT2

        @pl.when(j >= i)
        def _():
            xib = xnb_s[pl.ds(i * T2, T2), :]
            xjb = xnb_s[pl.ds(j * T2, T2), :]
            s = jax.lax.dot_general(xib, xjb, (((1,), (1,)), ((), ())),
                                    preferred_element_type=jnp.float32)
            colg = jax.lax.broadcasted_iota(jnp.int32, (T2, T2), 1)
            rowg = jax.lax.broadcasted_iota(jnp.int32, (T2, T2), 0)
            off_diag = (colg > rowg) | (j != i)
            sus = ((s > SUS_THR) & off_diag).astype(jnp.float32)
            v = jnp.sum(sus)
            cnt_sm[i * NT2 + j] += v
            cnt_sm[j * NT2 + i] += v
        return c

    jax.lax.fori_loop(0, NT2 * NT2, pair_body, 0)

    def dv_of(i):
        return dinv_s[pl.ds(i * T2, T2), :]

    def row_active(i):
        def rbody(j, r):
            return jnp.maximum(r, cnt_sm[i * NT2 + j])
        return jax.lax.fori_loop(0, NT2, rbody, jnp.float32(0.0))

    def exact_tile(i, j):
        rows = pl.ds(i * T2, T2)
        cols = pl.ds(j * T2, T2)
        xi = x_ref[rows, :] / nrm_ref[rows, :]
        xj = x_ref[cols, :] / nrm_ref[cols, :]
        s = jax.lax.dot_general(xi, xj, (((1,), (1,)), ((), ())),
                                preferred_element_type=jnp.float32)
        colg = jax.lax.broadcasted_iota(jnp.int32, (T2, T2), 1) + j * T2
        rowg = jax.lax.broadcasted_iota(jnp.int32, (T2, T2), 0) + i * T2
        return jnp.where((s > THR) & (colg != rowg), s, 0.0)

    def deg_dinv(i):
        rows = pl.ds(i * T2, T2)
        act = row_active(i)

        @pl.when(act > 0.5)
        def _():
            acc_s[...] = jnp.zeros((T2, H), jnp.float32)

            def dbody(j, carry):
                @pl.when(cnt_sm[i * NT2 + j] > 0.5)
                def _():
                    a = exact_tile(i, j)
                    acc_s[:, 0:1] += jnp.sum(a, axis=1, keepdims=True)
                return carry

            jax.lax.fori_loop(0, NT2, dbody, 0)
            deg = acc_s[:, 0:1]
            dinv_s[rows, :] = jnp.where(
                deg > 0, jax.lax.rsqrt(jnp.maximum(deg, 1e-12)), 0.0)

        @pl.when(act <= 0.5)
        def _():
            dinv_s[rows, :] = jnp.zeros((T2, 1), jnp.float32)

    def phase_a(i, h, wcat, v1_s):
        rows = pl.ds(i * T2, T2)
        bvv = jnp.dot(h, wcat[...], preferred_element_type=jnp.float32)
        b0, v1, v2 = bvv[:, :H], bvv[:, H:2 * H], bvv[:, 2 * H:]
        base_s[rows, :] = b0 - v2

        @pl.when(row_active(i) > 0.5)
        def _():
            v1_s[rows, :] = v1
            v2_s[rows, :] = dv_of(i) * v2

    def spmm_rows(i, src_s):
        acc_s[...] = jnp.zeros((T2, H), jnp.float32)

        def body(j, carry):
            @pl.when(cnt_sm[i * NT2 + j] > 0.5)
            def _():
                a = exact_tile(i, j)
                acc_s[...] += jnp.dot(a, src_s[pl.ds(j * T2, T2), :],
                                      preferred_element_type=jnp.float32)
            return carry

        jax.lax.fori_loop(0, NT2, body, 0)
        return -dv_of(i) * acc_s[...]

    def phase_b(i, v1_s):
        rows = pl.ds(i * T2, T2)
        act = row_active(i)

        @pl.when(act > 0.5)
        def _():
            m2 = spmm_rows(i, v2_s)
            v1_s[rows, :] = dv_of(i) * (v1_s[rows, :] + 2.0 * m2)

    def relu_into(i, v1_s, dst_s):
        rows = pl.ds(i * T2, T2)
        act = row_active(i)

        @pl.when(act > 0.5)
        def _():
            dst_s[rows, :] = jnp.maximum(
                base_s[rows, :] + spmm_rows(i, v1_s), 0.0)

        @pl.when(act <= 0.5)
        def _():
            dst_s[rows, :] = jnp.maximum(base_s[rows, :], 0.0)

    def head(i):
        rows = pl.ds(i * T2, T2)
        z = (jnp.dot(h1_s[rows, :], l1w[:H, :],
                     preferred_element_type=jnp.float32)
             + jnp.dot(h2_s[rows, :], l1w[H:2 * H, :],
                       preferred_element_type=jnp.float32)
             + jnp.dot(h3_s[rows, :], l1w[2 * H:, :],
                       preferred_element_type=jnp.float32)
             + l1b[...])
        z = jnp.maximum(z, 0.0)
        z = z * (_BN_SCALE * bg[...]) + bb[...]
        logit = (jnp.dot(z, l2w[...], preferred_element_type=jnp.float32)
                 + l2b[...])
        m = jnp.max(logit, axis=1, keepdims=True)
        e = jnp.exp(logit - m)
        out_ref[rows, :] = e / jnp.sum(e, axis=1, keepdims=True)

    def loop(fn):
        jax.lax.fori_loop(0, NT2, lambda i, c: (fn(i), c)[1], 0)

    def dega0(i):
        deg_dinv(i)
        phase_a(i, x_ref[pl.ds(i * T2, T2), :], wc0, v1a_s)

    loop(dega0)
    loop(lambda i: phase_b(i, v1a_s))

    def c0a1(i):
        relu_into(i, v1a_s, h1_s)
        phase_a(i, h1_s[pl.ds(i * T2, T2), :], wc1, v1b_s)

    loop(c0a1)
    loop(lambda i: phase_b(i, v1b_s))

    def c1a2(i):
        relu_into(i, v1b_s, h2_s)
        phase_a(i, h2_s[pl.ds(i * T2, T2), :], wc2, v1a_s)

    loop(c1a2)
    loop(lambda i: phase_b(i, v1a_s))

    def c2head(i):
        relu_into(i, v1a_s, h3_s)
        head(i)

    loop(c2head)


def kernel(x, w0_0, w0_1, w0_2, w1_0, w1_1, w1_2, w2_0, w2_1, w2_2,
           lin1_w, lin1_b, bn_gamma, bn_beta, lin2_w, lin2_b):
    f32 = jnp.float32
    full = pl.BlockSpec(memory_space=pltpu.VMEM)
    wc0 = jnp.concatenate([w0_0, w0_1, w0_2], axis=1)
    wc1 = jnp.concatenate([w1_0, w1_1, w1_2], axis=1)
    wc2 = jnp.concatenate([w2_0, w2_1, w2_2], axis=1)
    out = pl.pallas_call(
        _gcn_kernel,
        in_specs=[full] * 10,
        out_specs=full,
        out_shape=jax.ShapeDtypeStruct((N, 10), f32),
        scratch_shapes=[pltpu.VMEM((N, H), f32)] * 7
                       + [pltpu.VMEM((N, 1), f32)]
                       + [pltpu.VMEM((T2, H), f32)]
                       + [pltpu.VMEM((N, 1), f32),
                          pltpu.VMEM((N, D), jnp.bfloat16),
                          pltpu.SMEM((NT2 * NT2,), f32)],
    )(x, wc0, wc1, wc2,
      lin1_w, lin1_b.reshape(1, -1), bn_gamma.reshape(1, -1),
      bn_beta.reshape(1, -1), lin2_w, lin2_b.reshape(1, -1))
    return out
